# baseline (speedup 1.0000x reference)
import jax
import jax.numpy as jnp
from jax import lax
from jax.experimental import pallas as pl
from jax.experimental.pallas import tpu as pltpu
from jax.experimental.pallas import tpu_sc as plsc

B, S, D = 4, 8192, 1024
NC, NS = 2, 16
NW = NC * NS
ROWS_W = S // NW
CH = 16
NCHUNK = ROWS_W // CH
CHW = CH * D
LANES = 16
NVEC = CHW // LANES
NT = NCHUNK * B
NBUF = 5
LOOK = 3
CPR = D // LANES


def _pe_body(x_hbm, tbl_hbm, out_hbm, *scratch):
    xbuf = scratch[:NBUF]
    tblv = scratch[NBUF:NBUF + 2]
    isem = scratch[NBUF + 2:NBUF + 2 + NBUF]
    osem = scratch[NBUF + 2 + NBUF:NBUF + 2 + 2 * NBUF]
    tsem = scratch[NBUF + 2 + 2 * NBUF:]

    cid = lax.axis_index("c")
    sid = lax.axis_index("s")
    wid = sid * NC + cid
    row0 = wid * ROWS_W

    def in_copy(t):
        c, b = divmod(t, B)
        return pltpu.async_copy(
            x_hbm.at[b, pl.ds(row0 + c * CH, CH), :],
            xbuf[t % NBUF], isem[t % NBUF])

    def out_copy(t):
        c, b = divmod(t, B)
        return pltpu.async_copy(
            xbuf[t % NBUF],
            out_hbm.at[b, pl.ds(row0 + c * CH, CH), :], osem[t % NBUF])

    def tbl_copy(c):
        return pltpu.async_copy(
            tbl_hbm.at[pl.ds(row0 + c * CH, CH), :], tblv[c % 2], tsem[c % 2])

    in_d, out_d, tbl_d = {}, {}, {}
    for t in range(LOOK):
        in_d[t] = in_copy(t)
    tbl_d[0] = tbl_copy(0)
    tbl_d[1] = tbl_copy(1)

    for t in range(NT):
        c, b = divmod(t, B)
        ta = t + LOOK
        if ta < NT:
            if ta - NBUF >= 0:
                out_d[ta - NBUF].wait()
            in_d[ta] = in_copy(ta)
        if b == 0:
            if c >= 1 and c + 1 < NCHUNK:
                tbl_d[c + 1] = tbl_copy(c + 1)
            tbl_d[c].wait()
        in_d[t].wait()
        xb = xbuf[t % NBUF]
        tb = tblv[c % 2]

        @plsc.parallel_loop(0, NVEC, unroll=16)
        def _(i):
            r = i // CPR
            col = (i % CPR) * LANES
            plsc.addupdate(
                xb.at[r, pl.ds(col, LANES)],
                tb[r, pl.ds(col, LANES)],
            )

        out_d[t] = out_copy(t)

    for t in range(NT - NBUF, NT):
        out_d[t].wait()


_pe_call = pl.kernel(
    _pe_body,
    out_type=jax.ShapeDtypeStruct((B, S, D), jnp.float32),
    mesh=plsc.VectorSubcoreMesh(core_axis_name="c", subcore_axis_name="s"),
    scratch_types=(
        [pltpu.VMEM((CH, D), jnp.float32) for _ in range(NBUF + 2)]
        + [pltpu.SemaphoreType.DMA for _ in range(2 * NBUF + 2)]
    ),
)


@jax.jit
def kernel(x, position_matrix):
    return _pe_call(x, position_matrix)
